# baseline (speedup 1.0000x reference)
import functools

import jax
import jax.numpy as jnp
from jax.experimental import pallas as pl
from jax.experimental.pallas import tpu as pltpu
from jax.experimental.pallas import tpu_sc as plsc

_Q = 1024
_D = 16
_BK = 5000
_NKEYS = 100000


def _reduce_body(nsteps, y_ref, xt2_ref, m_ref, i_ref, m_scr, i_scr, xsq_scr):
    step = pl.program_id(0)

    @pl.when(step == 0)
    def _():
        xt2 = xt2_ref[...]
        xsq_scr[0, :] = jnp.sum(xt2 * xt2, axis=0) * 0.25

    y = y_ref[...]
    y_sq = jnp.sum(y * y, axis=1, keepdims=True)
    prod = jnp.dot(y, xt2_ref[...],
                   preferred_element_type=jnp.float32)
    d2 = (y_sq + xsq_scr[0, :][None, :]) + prod
    bm = jnp.min(d2, axis=0)
    ba = jnp.argmin(d2, axis=0).astype(jnp.int32)
    base = step * _BK

    @pl.when(step == 0)
    def _():
        m_scr[0, :] = bm
        i_scr[0, :] = ba

    @pl.when(step > 0)
    def _():
        cur_m = m_scr[0, :]
        better = bm < cur_m
        m_scr[0, :] = jnp.where(better, bm, cur_m)
        i_scr[0, :] = jnp.where(better, ba + base, i_scr[0, :])

    @pl.when(step == nsteps - 1)
    def _():
        m_ref[0, :] = m_scr[0, :]
        i_ref[0, :] = i_scr[0, :]


def _knn_reduce(y, xt2, *, interpret=False):
    nkeys = y.shape[0]
    nsteps = nkeys // _BK
    return pl.pallas_call(
        functools.partial(_reduce_body, nsteps),
        grid=(nsteps,),
        in_specs=[
            pl.BlockSpec((_BK, _D), lambda i: (i, 0)),
            pl.BlockSpec((_D, _Q), lambda i: (0, 0)),
        ],
        out_specs=[
            pl.BlockSpec((1, _Q), lambda i: (0, 0)),
            pl.BlockSpec((1, _Q), lambda i: (0, 0)),
        ],
        out_shape=[
            jax.ShapeDtypeStruct((1, _Q), jnp.float32),
            jax.ShapeDtypeStruct((1, _Q), jnp.int32),
        ],
        scratch_shapes=[
            pltpu.VMEM((1, _Q), jnp.float32),
            pltpu.VMEM((1, _Q), jnp.int32),
            pltpu.VMEM((1, _Q), jnp.float32),
        ],
        compiler_params=pltpu.CompilerParams(
            dimension_semantics=("arbitrary",),
        ),
        interpret=interpret,
    )(y, xt2)


def _sc_scatter_update(min_dists, nn_indices, upd_d, upd_i, start):
    start_arr = jnp.asarray(start, jnp.int32).reshape(1)
    mesh = plsc.ScalarSubcoreMesh(axis_name="core", num_cores=2)
    nrows, qrows = _NKEYS // 8, _Q // 8

    @functools.partial(
        pl.kernel,
        out_type=(
            jax.ShapeDtypeStruct((nrows, 8), jnp.float32),
            jax.ShapeDtypeStruct((nrows, 8), jnp.int32),
        ),
        mesh=mesh,
        scratch_types=[pltpu.SMEM((1,), jnp.int32),
                       pltpu.SemaphoreType.DMA],
    )
    def sc_scatter(md, ni, ud, ui, st, md_out, ni_out, st_smem, sem):
        core = jax.lax.axis_index("core")
        pltpu.async_copy(st, st_smem, sem).wait()
        srow = pl.multiple_of(st_smem[0] // 8, 8)

        @pl.when(core == 0)
        def _():
            pltpu.async_copy(md, md_out, sem).wait()
            pltpu.async_copy(ud, md_out.at[pl.ds(srow, qrows)], sem).wait()

        @pl.when(core == 1)
        def _():
            pltpu.async_copy(ni, ni_out, sem).wait()
            pltpu.async_copy(ui, ni_out.at[pl.ds(srow, qrows)], sem).wait()

    md_new, ni_new = sc_scatter(min_dists.reshape(nrows, 8),
                                nn_indices.reshape(nrows, 8),
                                upd_d.reshape(qrows, 8),
                                upd_i.reshape(qrows, 8), start_arr)
    return md_new.reshape(_NKEYS), ni_new.reshape(_NKEYS)


def kernel(x, x_idx_start, y, y_idx_start, min_dists, nn_indices):
    xt2 = (-2.0 * x.reshape(_Q, _D)).T
    m, i = _knn_reduce(y, xt2)
    old = jax.lax.dynamic_slice(min_dists, (x_idx_start,), (_Q,))
    new_d = jnp.sqrt(jnp.maximum(m.reshape(_Q), 0.0))
    upd_d = jnp.minimum(new_d, old)
    upd_i = (i.reshape(_Q) + y_idx_start).astype(nn_indices.dtype)
    return _sc_scatter_update(min_dists, nn_indices, upd_d, upd_i,
                              x_idx_start)


# speedup vs baseline: 4.4756x; 4.4756x over previous
import functools

import jax
import jax.numpy as jnp
from jax.experimental import pallas as pl
from jax.experimental.pallas import tpu as pltpu

_Q = 1024
_D = 16
_BK = 10000
_NKEYS = 100000


def _reduce_body(nsteps, y_ref, xt2_ref, m_ref, i_ref, m_scr, i_scr, xsq_scr):
    step = pl.program_id(0)

    @pl.when(step == 0)
    def _():
        xt2 = xt2_ref[...]
        xsq_scr[0, :] = jnp.sum(xt2 * xt2, axis=0) * 0.25

    y = y_ref[...]
    y_sq = jnp.sum(y * y, axis=1, keepdims=True)
    prod = jnp.dot(y, xt2_ref[...],
                   preferred_element_type=jnp.float32)
    d2 = (y_sq + xsq_scr[0, :][None, :]) + prod
    bm = jnp.min(d2, axis=0)
    ba = jnp.argmin(d2, axis=0).astype(jnp.int32)
    base = step * _BK

    @pl.when(step == 0)
    def _():
        m_scr[0, :] = bm
        i_scr[0, :] = ba

    @pl.when(step > 0)
    def _():
        cur_m = m_scr[0, :]
        better = bm < cur_m
        m_scr[0, :] = jnp.where(better, bm, cur_m)
        i_scr[0, :] = jnp.where(better, ba + base, i_scr[0, :])

    @pl.when(step == nsteps - 1)
    def _():
        m_ref[0, :] = m_scr[0, :]
        i_ref[0, :] = i_scr[0, :]


def _knn_reduce(y, xt2, *, interpret=False):
    nkeys = y.shape[0]
    nsteps = nkeys // _BK
    return pl.pallas_call(
        functools.partial(_reduce_body, nsteps),
        grid=(nsteps,),
        in_specs=[
            pl.BlockSpec((_BK, _D), lambda i: (i, 0)),
            pl.BlockSpec((_D, _Q), lambda i: (0, 0)),
        ],
        out_specs=[
            pl.BlockSpec((1, _Q), lambda i: (0, 0)),
            pl.BlockSpec((1, _Q), lambda i: (0, 0)),
        ],
        out_shape=[
            jax.ShapeDtypeStruct((1, _Q), jnp.float32),
            jax.ShapeDtypeStruct((1, _Q), jnp.int32),
        ],
        scratch_shapes=[
            pltpu.VMEM((1, _Q), jnp.float32),
            pltpu.VMEM((1, _Q), jnp.int32),
            pltpu.VMEM((1, _Q), jnp.float32),
        ],
        compiler_params=pltpu.CompilerParams(
            dimension_semantics=("arbitrary",),
        ),
        interpret=interpret,
    )(y, xt2)


def kernel(x, x_idx_start, y, y_idx_start, min_dists, nn_indices):
    xt2 = (-2.0 * x.reshape(_Q, _D)).T
    m, i = _knn_reduce(y, xt2)
    old = jax.lax.dynamic_slice(min_dists, (x_idx_start,), (_Q,))
    new_d = jnp.sqrt(jnp.maximum(m.reshape(_Q), 0.0))
    upd_d = jnp.minimum(new_d, old)
    upd_i = (i.reshape(_Q) + y_idx_start).astype(nn_indices.dtype)
    min_dists_new = jax.lax.dynamic_update_slice(min_dists, upd_d,
                                                 (x_idx_start,))
    nn_indices_new = jax.lax.dynamic_update_slice(nn_indices, upd_i,
                                                  (x_idx_start,))
    return (min_dists_new, nn_indices_new)
